# baseline (speedup 1.0000x reference)
import jax
import jax.numpy as jnp
from jax.experimental import pallas as pl

G = 16
RATIO = 0.8


def _topk_perm(score, batch_ids, num_groups, ratio):
    n = score.shape[0]
    pos = jnp.arange(n, dtype=jnp.int32)
    valid = batch_ids < num_groups
    counts = jax.ops.segment_sum(jnp.ones((n,), jnp.int32), batch_ids, num_segments=num_groups)
    k = jnp.ceil(ratio * counts.astype(jnp.float32)).astype(jnp.int32)
    start = jnp.concatenate([jnp.zeros((1,), jnp.int32), jnp.cumsum(counts)])
    order = jnp.lexsort((-score, batch_ids)).astype(jnp.int32)
    bclip = jnp.minimum(batch_ids, num_groups - 1)
    rank = pos - start[bclip]
    keep = valid & (rank < k[bclip])
    dest = jnp.where(keep, jnp.cumsum(keep.astype(jnp.int32)) - 1, n)
    perm = jnp.zeros((n,), jnp.int32).at[dest].set(order, mode='drop')
    kcount = jnp.sum(keep.astype(jnp.int32))
    return perm, pos < kcount


def _filter_edges(ei, perm, perm_valid, num_nodes):
    new_id = jnp.full((num_nodes,), -1, jnp.int32).at[
        jnp.where(perm_valid, perm, num_nodes)].set(
        jnp.arange(num_nodes, dtype=jnp.int32), mode='drop')
    src, dst = ei[0], ei[1]
    ns, nd = new_id[src], new_id[dst]
    emask = (ns >= 0) & (nd >= 0)
    new_ei = jnp.stack([jnp.where(emask, ns, 0), jnp.where(emask, nd, num_nodes)]).astype(jnp.int32)
    return new_ei


def _graph_conv(x, ei, ew, W_rel, b_rel, W_root):
    src, dst = ei[0], ei[1]
    msg = x[src] * ew[:, None]
    agg = jax.ops.segment_sum(msg, dst, num_segments=x.shape[0])
    return agg @ W_rel + b_rel + x @ W_root


def _gmp(x, batch, num_segments):
    return jax.ops.segment_max(x, batch, num_segments=num_segments)


def _gap(x, batch, num_segments):
    s = jax.ops.segment_sum(x, batch, num_segments=num_segments)
    cnt = jax.ops.segment_sum(jnp.ones((x.shape[0],), x.dtype), batch, num_segments=num_segments)
    return s / jnp.clip(cnt, 1.0)[:, None]


def kernel(x, edge_index, edge_attr, batch, W_rel1, b_rel1, W_root1, p1,
           W_rel2, b_rel2, W_root2, p2, W_l1, b_l1, W_l2, b_l2, W_l3, b_l3):
    num_graphs = G
    h = jax.nn.relu(_graph_conv(x, edge_index, edge_attr, W_rel1, b_rel1, W_root1))
    score1 = (h @ p1) / jnp.linalg.norm(p1)
    perm1, valid1 = _topk_perm(score1, batch, num_graphs, RATIO)
    ei1 = _filter_edges(edge_index, perm1, valid1, h.shape[0])
    h = h[perm1] * jnp.tanh(score1[perm1])[:, None]
    ew1 = edge_attr
    batch1 = jnp.where(valid1, batch[perm1], num_graphs).astype(jnp.int32)
    x1 = jnp.concatenate([_gmp(h, batch1, num_graphs), _gap(h, batch1, num_graphs)], axis=1)
    h2 = jax.nn.relu(_graph_conv(h, ei1, ew1, W_rel2, b_rel2, W_root2))
    score2 = (h2 @ p2) / jnp.linalg.norm(p2)
    perm2, valid2 = _topk_perm(score2, batch1, num_graphs, RATIO)
    h2 = h2[perm2] * jnp.tanh(score2[perm2])[:, None]
    batch2 = jnp.where(valid2, batch1[perm2], num_graphs).astype(jnp.int32)
    x2 = jnp.concatenate([_gmp(h2, batch2, num_graphs), _gap(h2, batch2, num_graphs)], axis=1)
    z = x1 + x2
    z = jax.nn.relu(z @ W_l1 + b_l1)
    z = jax.nn.relu(z @ W_l2 + b_l2)
    z = jax.nn.log_softmax(z @ W_l3 + b_l3, axis=-1)
    return z


# speedup vs baseline: 6.5412x; 6.5412x over previous
import functools

import jax
import jax.numpy as jnp
from jax import lax
from jax.experimental import pallas as pl
from jax.experimental.pallas import tpu as pltpu
from jax.experimental.pallas import tpu_sc as plsc

N = 10000
E = 320000
G = 16
RATIO = 0.8
NEG = -jnp.inf

_EC = 256
_NCHUNK = E // _EC
_NW = 32
_CHUNK_ITERS = (_NCHUNK + _NW - 1) // _NW
_XPC = 200
_NXP = N // _XPC


def _make_edge_agg(D):
    nv = D // 16
    mesh = plsc.VectorSubcoreMesh(core_axis_name="c", subcore_axis_name="s")

    @functools.partial(
        pl.kernel,
        mesh=mesh,
        out_type=jax.ShapeDtypeStruct((2, N, D), jnp.float32),
        compiler_params=pltpu.CompilerParams(use_tc_tiling_on_sc=False),
        scratch_types=[
            pltpu.VMEM((_EC,), jnp.int32),
            pltpu.VMEM((_EC,), jnp.int32),
            pltpu.VMEM((_EC, 16), jnp.float32),
            pltpu.VMEM((_EC, D), jnp.float32),
            pltpu.VMEM_SHARED((N, D), jnp.float32),
            pltpu.SemaphoreType.DMA,
        ],
    )
    def agg(vals, srci, dsti, eww, out, src_v, dst_v, ew_v, rows_v, acc, sem):
        cid = lax.axis_index("c")
        sid = lax.axis_index("s")
        wid = sid * 2 + cid

        def zrow(r, carry):
            for v in range(nv):
                rows_v[r, pl.ds(v * 16, 16)] = jnp.zeros((16,), jnp.float32)
            return carry
        lax.fori_loop(0, _EC, zrow, 0)
        for qq in range((_NXP + 15) // 16):
            q = qq * 16 + sid

            @pl.when(q < _NXP)
            def _():
                r0 = pl.multiple_of(q * _XPC, _XPC)
                pltpu.sync_copy(rows_v.at[pl.ds(0, _XPC)], acc.at[pl.ds(r0, _XPC)])
        plsc.subcore_barrier()

        def ebody(e, carry):
            w = ew_v[e, :]
            for v in range(nv):
                sl = pl.ds(v * 16, 16)
                rows_v[e, sl] = rows_v[e, sl] * w
            return carry

        def cbody(i, carry):
            j = wid + i * _NW

            @pl.when(j < _NCHUNK)
            def _():
                base = pl.multiple_of(j * _EC, _EC)
                pltpu.sync_copy(srci.at[pl.ds(base, _EC)], src_v)
                pltpu.sync_copy(dsti.at[pl.ds(base, _EC)], dst_v)
                pltpu.sync_copy(eww.at[pl.ds(base, _EC)], ew_v)
                pltpu.async_copy(vals.at[src_v], rows_v, sem).wait()
                lax.fori_loop(0, _EC, ebody, 0)
                pltpu.sync_copy(rows_v, acc.at[dst_v], add=True)
            return carry

        lax.fori_loop(0, _CHUNK_ITERS, cbody, 0)
        plsc.subcore_barrier()

        for qq in range((_NXP + 15) // 16):
            q = qq * 16 + sid

            @pl.when(q < _NXP)
            def _():
                r0 = pl.multiple_of(q * _XPC, _XPC)
                pltpu.sync_copy(acc.at[pl.ds(r0, _XPC)], out.at[cid, pl.ds(r0, _XPC)])

    return agg


_agg64 = _make_edge_agg(64)
_agg32 = _make_edge_agg(32)


def _rank_in_graph(score, batch, num_groups):
    n = score.shape[0]
    order = jnp.lexsort((-score, batch)).astype(jnp.int32)
    counts = jax.ops.segment_sum(jnp.ones((n,), jnp.int32), batch, num_segments=num_groups)
    start = jnp.concatenate([jnp.zeros((1,), jnp.int32), jnp.cumsum(counts)])
    pos = jnp.arange(n, dtype=jnp.int32)
    rank_sorted = pos - start[jnp.minimum(batch[order], num_groups - 1)]
    rank = jnp.zeros((n,), jnp.int32).at[order].set(rank_sorted)
    return rank, counts


def kernel(x, edge_index, edge_attr, batch, W_rel1, b_rel1, W_root1, p1,
           W_rel2, b_rel2, W_root2, p2, W_l1, b_l1, W_l2, b_l2, W_l3, b_l3):
    src, dst = edge_index[0], edge_index[1]
    ew = edge_attr

    ewx = jnp.broadcast_to(ew[:, None], (E, 16))
    Pa = _agg64(x[:, :64], src, dst, ewx)
    Pb = _agg64(x[:, 64:], src, dst, ewx)
    agg1 = jnp.concatenate([Pa[0] + Pa[1], Pb[0] + Pb[1]], axis=1)
    h = jax.nn.relu(agg1 @ W_rel1 + b_rel1 + x @ W_root1)
    s1 = (h @ p1) / jnp.linalg.norm(p1)

    rank1, counts = _rank_in_graph(s1, batch, G)
    k1 = jnp.ceil(RATIO * counts.astype(jnp.float32)).astype(jnp.int32)
    keep1 = (batch < G) & (rank1 < k1[jnp.minimum(batch, G - 1)])
    g1 = jnp.tanh(s1)
    h1 = jnp.where(keep1[:, None], h * g1[:, None], 0.0)
    x1max = jax.ops.segment_max(jnp.where(keep1[:, None], h * g1[:, None], NEG), batch, num_segments=G)
    x1mean = jax.ops.segment_sum(h1, batch, num_segments=G) / jnp.clip(k1.astype(jnp.float32), 1.0)[:, None]
    x1 = jnp.concatenate([x1max, x1mean], axis=1)

    Q = _agg32(h1, src, dst, ewx)
    agg2 = Q[0] + Q[1]
    h2 = jax.nn.relu(agg2 @ W_rel2 + b_rel2 + h1 @ W_root2)
    s2 = (h2 @ p2) / jnp.linalg.norm(p2)

    s2m = jnp.where(keep1, s2, NEG)
    rank2, _ = _rank_in_graph(s2m, batch, G)
    k2 = jnp.ceil(RATIO * k1.astype(jnp.float32)).astype(jnp.int32)
    keep2 = keep1 & (rank2 < k2[jnp.minimum(batch, G - 1)])
    g2 = jnp.tanh(s2)
    h2m = jnp.where(keep2[:, None], h2 * g2[:, None], 0.0)
    x2max = jax.ops.segment_max(jnp.where(keep2[:, None], h2 * g2[:, None], NEG), batch, num_segments=G)
    x2mean = jax.ops.segment_sum(h2m, batch, num_segments=G) / jnp.clip(k2.astype(jnp.float32), 1.0)[:, None]
    x2 = jnp.concatenate([x2max, x2mean], axis=1)

    z = x1 + x2
    z = jax.nn.relu(z @ W_l1 + b_l1)
    z = jax.nn.relu(z @ W_l2 + b_l2)
    z = jax.nn.log_softmax(z @ W_l3 + b_l3, axis=-1)
    return z


# speedup vs baseline: 7.6644x; 1.1717x over previous
import functools

import jax
import jax.numpy as jnp
from jax import lax
from jax.experimental import pallas as pl
from jax.experimental.pallas import tpu as pltpu
from jax.experimental.pallas import tpu_sc as plsc

N = 10000
E = 320000
G = 16
RATIO = 0.8
NEG = -jnp.inf

_EC = 800
_NCHUNK = E // _EC
_NW = 32
_CHUNK_ITERS = (_NCHUNK + _NW - 1) // _NW
_EU = 8
_XPC = 200
_NXP = N // _XPC


def _make_edge_agg(D):
    nv = D // 16
    mesh = plsc.VectorSubcoreMesh(core_axis_name="c", subcore_axis_name="s")

    @functools.partial(
        pl.kernel,
        mesh=mesh,
        out_type=jax.ShapeDtypeStruct((2, N, D), jnp.float32),
        compiler_params=pltpu.CompilerParams(use_tc_tiling_on_sc=False),
        scratch_types=[
            pltpu.VMEM((_EC,), jnp.int32),
            pltpu.VMEM((_EC,), jnp.int32),
            pltpu.VMEM((_EC, 16), jnp.float32),
            pltpu.VMEM((_EC, D), jnp.float32),
            pltpu.VMEM_SHARED((N, D), jnp.float32),
            pltpu.SemaphoreType.DMA,
        ],
    )
    def agg(vals, srci, dsti, eww, out, src_v, dst_v, ew_v, rows_v, acc, sem):
        cid = lax.axis_index("c")
        sid = lax.axis_index("s")
        wid = sid * 2 + cid

        def zrow(r, carry):
            for v in range(nv):
                rows_v[r, pl.ds(v * 16, 16)] = jnp.zeros((16,), jnp.float32)
            return carry
        lax.fori_loop(0, _XPC, zrow, 0)
        for qq in range((_NXP + 15) // 16):
            q = qq * 16 + sid

            @pl.when(q < _NXP)
            def _():
                r0 = pl.multiple_of(q * _XPC, _XPC)
                pltpu.sync_copy(rows_v.at[pl.ds(0, _XPC)], acc.at[pl.ds(r0, _XPC)])
        plsc.subcore_barrier()

        def ebody(eu, carry):
            for u in range(_EU):
                e = eu * _EU + u
                w = ew_v[e, :]
                for v in range(nv):
                    sl = pl.ds(v * 16, 16)
                    rows_v[e, sl] = rows_v[e, sl] * w
            return carry

        def cbody(i, carry):
            j = wid + i * _NW

            @pl.when(j < _NCHUNK)
            def _():
                base = pl.multiple_of(j * _EC, _EC)
                pltpu.sync_copy(srci.at[pl.ds(base, _EC)], src_v)
                pltpu.sync_copy(dsti.at[pl.ds(base, _EC)], dst_v)
                pltpu.sync_copy(eww.at[pl.ds(base, _EC)], ew_v)
                pltpu.async_copy(vals.at[src_v], rows_v, sem).wait()
                lax.fori_loop(0, _EC // _EU, ebody, 0)
                pltpu.sync_copy(rows_v, acc.at[dst_v], add=True)
            return carry

        lax.fori_loop(0, _CHUNK_ITERS, cbody, 0)
        plsc.subcore_barrier()

        for qq in range((_NXP + 15) // 16):
            q = qq * 16 + sid

            @pl.when(q < _NXP)
            def _():
                r0 = pl.multiple_of(q * _XPC, _XPC)
                pltpu.sync_copy(acc.at[pl.ds(r0, _XPC)], out.at[cid, pl.ds(r0, _XPC)])

    return agg


_agg64 = _make_edge_agg(64)
_agg32 = _make_edge_agg(32)


def _rank_in_graph(score, batch, num_groups):
    n = score.shape[0]
    order = jnp.lexsort((-score, batch)).astype(jnp.int32)
    counts = jax.ops.segment_sum(jnp.ones((n,), jnp.int32), batch, num_segments=num_groups)
    start = jnp.concatenate([jnp.zeros((1,), jnp.int32), jnp.cumsum(counts)])
    pos = jnp.arange(n, dtype=jnp.int32)
    rank_sorted = pos - start[jnp.minimum(batch[order], num_groups - 1)]
    rank = jnp.zeros((n,), jnp.int32).at[order].set(rank_sorted)
    return rank, counts


def kernel(x, edge_index, edge_attr, batch, W_rel1, b_rel1, W_root1, p1,
           W_rel2, b_rel2, W_root2, p2, W_l1, b_l1, W_l2, b_l2, W_l3, b_l3):
    src, dst = edge_index[0], edge_index[1]
    ew = edge_attr

    ewx = jnp.broadcast_to(ew[:, None], (E, 16))
    Pa = _agg64(x[:, :64], src, dst, ewx)
    Pb = _agg64(x[:, 64:], src, dst, ewx)
    agg1 = jnp.concatenate([Pa[0] + Pa[1], Pb[0] + Pb[1]], axis=1)
    h = jax.nn.relu(agg1 @ W_rel1 + b_rel1 + x @ W_root1)
    s1 = (h @ p1) / jnp.linalg.norm(p1)

    rank1, counts = _rank_in_graph(s1, batch, G)
    k1 = jnp.ceil(RATIO * counts.astype(jnp.float32)).astype(jnp.int32)
    keep1 = (batch < G) & (rank1 < k1[jnp.minimum(batch, G - 1)])
    g1 = jnp.tanh(s1)
    h1 = jnp.where(keep1[:, None], h * g1[:, None], 0.0)
    x1max = jax.ops.segment_max(jnp.where(keep1[:, None], h * g1[:, None], NEG), batch, num_segments=G)
    x1mean = jax.ops.segment_sum(h1, batch, num_segments=G) / jnp.clip(k1.astype(jnp.float32), 1.0)[:, None]
    x1 = jnp.concatenate([x1max, x1mean], axis=1)

    Q = _agg32(h1, src, dst, ewx)
    agg2 = Q[0] + Q[1]
    h2 = jax.nn.relu(agg2 @ W_rel2 + b_rel2 + h1 @ W_root2)
    s2 = (h2 @ p2) / jnp.linalg.norm(p2)

    s2m = jnp.where(keep1, s2, NEG)
    rank2, _ = _rank_in_graph(s2m, batch, G)
    k2 = jnp.ceil(RATIO * k1.astype(jnp.float32)).astype(jnp.int32)
    keep2 = keep1 & (rank2 < k2[jnp.minimum(batch, G - 1)])
    g2 = jnp.tanh(s2)
    h2m = jnp.where(keep2[:, None], h2 * g2[:, None], 0.0)
    x2max = jax.ops.segment_max(jnp.where(keep2[:, None], h2 * g2[:, None], NEG), batch, num_segments=G)
    x2mean = jax.ops.segment_sum(h2m, batch, num_segments=G) / jnp.clip(k2.astype(jnp.float32), 1.0)[:, None]
    x2 = jnp.concatenate([x2max, x2mean], axis=1)

    z = x1 + x2
    z = jax.nn.relu(z @ W_l1 + b_l1)
    z = jax.nn.relu(z @ W_l2 + b_l2)
    z = jax.nn.log_softmax(z @ W_l3 + b_l3, axis=-1)
    return z
